# baseline (speedup 1.0000x reference)
import functools

import jax
import jax.numpy as jnp
import numpy as np
from jax.experimental import pallas as pl
from jax.experimental.pallas import tpu as pltpu

EPS = 1e-5
_HALO = 64


def _compiler_params(ndims=1):
    return pltpu.CompilerParams(
        dimension_semantics=("parallel",) + ("arbitrary",) * (ndims - 1),
        vmem_limit_bytes=64 * 1024 * 1024,
    )


def _row_tile(nhw, target):
    for t in range(min(target, nhw), 7, -1):
        if nhw % t == 0 and t % 8 == 0:
            return t
    return nhw


def _stats_rows(y):
    return jnp.concatenate(
        [jnp.sum(y, axis=0, keepdims=True),
         jnp.sum(y * y, axis=0, keepdims=True)], axis=0)


def _fold_bn(stats, g, b, count, paired=False):
    s = jnp.sum(stats, axis=0)
    if paired:
        c = s.shape[1] // 2
        s = s[:, :c] + s[:, c:]
    mean = s[0:1] / count
    var = jnp.maximum(s[1:2] / count - mean * mean, 0.0)
    scale = g * jax.lax.rsqrt(var + EPS)
    shift = b - mean * scale
    return scale, shift


def _tile2(v):
    return jnp.concatenate([v, v], axis=1)


def _block_diag2(m):
    z = jnp.zeros(m.shape, m.dtype)
    return jnp.concatenate(
        [jnp.concatenate([m, z], axis=2),
         jnp.concatenate([z, m], axis=2)], axis=1)


def _conv1_kernel(xa_ref, xb_ref, w1_ref, t1_ref, s_ref):
    w1b = w1_ref[...].astype(jnp.bfloat16)
    ya = jnp.dot(xa_ref[...].astype(jnp.bfloat16), w1b,
                 preferred_element_type=jnp.float32)
    yb = jnp.dot(xb_ref[...].astype(jnp.bfloat16), w1b,
                 preferred_element_type=jnp.float32)
    s_ref[0] = _stats_rows(ya) + _stats_rows(yb)
    t1_ref[0] = jnp.concatenate(
        [ya.astype(jnp.bfloat16), yb.astype(jnp.bfloat16)], axis=1)


def _conv2_kernel(w, count, t1_ref, st1_ref, g1_ref, b1_ref, w2_ref,
                  mask_ref, t2_ref, s_ref, pad_ref):
    hw, lanes = t1_ref.shape[1], t1_ref.shape[2]
    p = _HALO
    scale, shift = _fold_bn(st1_ref[...], g1_ref[...], b1_ref[...], count)
    a = jnp.maximum(t1_ref[0].astype(jnp.float32) * _tile2(scale)
                    + _tile2(shift), 0.0)

    pad_ref[0:p, :] = jnp.zeros((p, lanes), jnp.bfloat16)
    pad_ref[p + hw:, :] = jnp.zeros((p, lanes), jnp.bfloat16)
    pad_ref[p:p + hw, :] = a.astype(jnp.bfloat16)

    w2d = _block_diag2(w2_ref[...].astype(jnp.bfloat16))
    acc = jnp.zeros((hw, lanes), jnp.float32)
    for k in range(9):
        dy, dx = k // 3 - 1, k % 3 - 1
        tap = pad_ref[p + dy * w + dx:p + dy * w + dx + hw, :]
        if dx == -1:
            tap = tap * mask_ref[:, 0:1]
        elif dx == 1:
            tap = tap * mask_ref[:, 1:2]
        acc = acc + jnp.dot(tap, w2d[k],
                            preferred_element_type=jnp.float32)
    t2_ref[0] = acc.astype(jnp.bfloat16)
    s_ref[0] = _stats_rows(acc)


def _conv3_stats_kernel(count, t2_ref, st2_ref, g2_ref, b2_ref, w3_ref,
                        s_ref):
    scale, shift = _fold_bn(st2_ref[...], g2_ref[...], b2_ref[...], count,
                            paired=True)
    a = jnp.maximum(t2_ref[0].astype(jnp.float32) * _tile2(scale)
                    + _tile2(shift), 0.0)
    w3d = _block_diag2(w3_ref[...].astype(jnp.bfloat16)[None])[0]
    y = jnp.dot(a.astype(jnp.bfloat16), w3d,
                preferred_element_type=jnp.float32)
    s_ref[0] = _stats_rows(y)


def _conv3_apply_kernel(cmid, count, t2_ref, st2_ref, g2_ref, b2_ref,
                        w3_ref, st3a_ref, st3b_ref, g3_ref, b3_ref, x_ref,
                        o_ref):
    scale2, shift2 = _fold_bn(st2_ref[...], g2_ref[...], b2_ref[...], count,
                              paired=True)
    s3 = jnp.sum(st3a_ref[...] + st3b_ref[...], axis=0)
    mean3 = s3[0:1] / count
    var3 = jnp.maximum(s3[1:2] / count - mean3 * mean3, 0.0)
    scale3 = g3_ref[...] * jax.lax.rsqrt(var3 + EPS)
    shift3 = b3_ref[...] - mean3 * scale3
    a = jnp.maximum(t2_ref[0].astype(jnp.float32) * _tile2(scale2)
                    + _tile2(shift2), 0.0)
    ab = a.astype(jnp.bfloat16)
    a2 = jnp.concatenate([ab[:, :cmid], ab[:, cmid:]], axis=0)
    y = jnp.dot(a2, w3_ref[...].astype(jnp.bfloat16),
                preferred_element_type=jnp.float32)
    o_ref[...] = jnp.maximum(y * scale3 + shift3 + x_ref[...], 0.0)


def kernel(x, w1, g1, b1, w2, g2, b2, w3, g3, b3):
    n, cin, h, w = x.shape
    cmid = w1.shape[1]
    cout = w3.shape[1]
    nhw = n * h * w
    npair = n // 2
    hw = h * w
    count = float(nhw)
    cp = _compiler_params()

    x_flat = jnp.transpose(x, (0, 2, 3, 1)).reshape(nhw, cin)

    col = np.arange(hw, dtype=np.int32) % w
    masks = jnp.asarray(
        np.stack([(col != 0), (col != w - 1)], axis=1).astype(np.float32),
        dtype=jnp.bfloat16)

    aff_mid = pl.BlockSpec((1, cmid), lambda i: (0, 0))
    st1_full = pl.BlockSpec((n, 2, cmid), lambda i: (0, 0, 0))
    st2_full = pl.BlockSpec((npair, 2, 2 * cmid), lambda i: (0, 0, 0))

    hw2 = hw // 2
    t1, s1 = pl.pallas_call(
        _conv1_kernel,
        grid=(npair, 2),
        in_specs=[pl.BlockSpec((hw2, cin), lambda i, j: (4 * i + j, 0)),
                  pl.BlockSpec((hw2, cin), lambda i, j: (4 * i + 2 + j, 0)),
                  pl.BlockSpec((cin, cmid), lambda i, j: (0, 0))],
        out_specs=(pl.BlockSpec((1, hw2, 2 * cmid),
                                lambda i, j: (i, j, 0)),
                   pl.BlockSpec((1, 2, cmid), lambda i, j: (2 * i + j, 0, 0))),
        out_shape=(jax.ShapeDtypeStruct((npair, hw, 2 * cmid),
                                        jnp.bfloat16),
                   jax.ShapeDtypeStruct((n, 2, cmid), jnp.float32)),
        compiler_params=_compiler_params(2),
    )(x_flat, x_flat, w1)

    t2, s2 = pl.pallas_call(
        functools.partial(_conv2_kernel, w, count),
        grid=(npair,),
        in_specs=[pl.BlockSpec((1, hw, 2 * cmid), lambda i: (i, 0, 0)),
                  st1_full, aff_mid, aff_mid,
                  pl.BlockSpec((9, cmid, cmid), lambda i: (0, 0, 0)),
                  pl.BlockSpec((hw, 2), lambda i: (0, 0))],
        out_specs=(pl.BlockSpec((1, hw, 2 * cmid), lambda i: (i, 0, 0)),
                   pl.BlockSpec((1, 2, 2 * cmid), lambda i: (i, 0, 0))),
        out_shape=(jax.ShapeDtypeStruct((npair, hw, 2 * cmid),
                                        jnp.bfloat16),
                   jax.ShapeDtypeStruct((npair, 2, 2 * cmid), jnp.float32)),
        scratch_shapes=[pltpu.VMEM((hw + 2 * _HALO, 2 * cmid),
                                   jnp.bfloat16)],
        compiler_params=cp,
    )(t1, s1, g1, b1, w2, masks)

    s3 = pl.pallas_call(
        functools.partial(_conv3_stats_kernel, count),
        grid=(npair,),
        in_specs=[pl.BlockSpec((1, hw, 2 * cmid), lambda i: (i, 0, 0)),
                  st2_full, aff_mid, aff_mid,
                  pl.BlockSpec((cmid, cout), lambda i: (0, 0))],
        out_specs=pl.BlockSpec((1, 2, 2 * cout), lambda i: (i, 0, 0)),
        out_shape=jax.ShapeDtypeStruct((npair, 2, 2 * cout), jnp.float32),
        compiler_params=cp,
    )(t2, s2, g2, b2, w3)

    ch = cout // 2
    out = pl.pallas_call(
        functools.partial(_conv3_apply_kernel, cmid, count),
        grid=(npair, 2),
        in_specs=[pl.BlockSpec((1, hw, 2 * cmid), lambda i, j: (i, 0, 0)),
                  pl.BlockSpec((npair, 2, 2 * cmid),
                               lambda i, j: (0, 0, 0)),
                  pl.BlockSpec((1, cmid), lambda i, j: (0, 0)),
                  pl.BlockSpec((1, cmid), lambda i, j: (0, 0)),
                  pl.BlockSpec((cmid, ch), lambda i, j: (0, j)),
                  pl.BlockSpec((npair, 2, ch), lambda i, j: (0, 0, j)),
                  pl.BlockSpec((npair, 2, ch), lambda i, j: (0, 0, 2 + j)),
                  pl.BlockSpec((1, ch), lambda i, j: (0, j)),
                  pl.BlockSpec((1, ch), lambda i, j: (0, j)),
                  pl.BlockSpec((2 * hw, ch), lambda i, j: (i, j))],
        out_specs=pl.BlockSpec((2 * hw, ch), lambda i, j: (i, j)),
        out_shape=jax.ShapeDtypeStruct((nhw, cout), jnp.float32),
        compiler_params=_compiler_params(2),
    )(t2, s2, g2, b2, w3, s3, s3, g3, b3, x_flat)

    return jnp.transpose(out.reshape(n, h, w, cout), (0, 3, 1, 2))


# speedup vs baseline: 1.0226x; 1.0226x over previous
import functools

import jax
import jax.numpy as jnp
import numpy as np
from jax.experimental import pallas as pl
from jax.experimental.pallas import tpu as pltpu

EPS = 1e-5
_HALO = 64


def _compiler_params(ndims=1):
    return pltpu.CompilerParams(
        dimension_semantics=("parallel",) + ("arbitrary",) * (ndims - 1),
        vmem_limit_bytes=64 * 1024 * 1024,
    )


def _row_tile(nhw, target):
    for t in range(min(target, nhw), 7, -1):
        if nhw % t == 0 and t % 8 == 0:
            return t
    return nhw


def _stats_rows(y):
    return jnp.concatenate(
        [jnp.sum(y, axis=0, keepdims=True),
         jnp.sum(y * y, axis=0, keepdims=True)], axis=0)


def _fold_bn(stats, g, b, count, paired=False):
    s = jnp.sum(stats, axis=0)
    if paired:
        c = s.shape[1] // 2
        s = s[:, :c] + s[:, c:]
    mean = s[0:1] / count
    var = jnp.maximum(s[1:2] / count - mean * mean, 0.0)
    scale = g * jax.lax.rsqrt(var + EPS)
    shift = b - mean * scale
    return scale, shift


def _tile2(v):
    return jnp.concatenate([v, v], axis=1)


def _block_diag2(m):
    z = jnp.zeros(m.shape, m.dtype)
    return jnp.concatenate(
        [jnp.concatenate([m, z], axis=2),
         jnp.concatenate([z, m], axis=2)], axis=1)


def _conv1_kernel(xa_ref, xb_ref, w1_ref, t1_ref, s_ref):
    w1b = w1_ref[...].astype(jnp.bfloat16)
    ya = jnp.dot(xa_ref[...].astype(jnp.bfloat16), w1b,
                 preferred_element_type=jnp.float32)
    yb = jnp.dot(xb_ref[...].astype(jnp.bfloat16), w1b,
                 preferred_element_type=jnp.float32)
    s_ref[0] = _stats_rows(ya) + _stats_rows(yb)
    t1_ref[0] = jnp.concatenate(
        [ya.astype(jnp.bfloat16), yb.astype(jnp.bfloat16)], axis=1)


def _conv2_kernel(w, count, t1_ref, st1_ref, g1_ref, b1_ref, w2_ref,
                  mask_ref, t2_ref, s_ref, pad_ref):
    hw, lanes = t1_ref.shape[1], t1_ref.shape[2]
    p = _HALO
    scale, shift = _fold_bn(st1_ref[...], g1_ref[...], b1_ref[...], count)
    a = jnp.maximum(t1_ref[0].astype(jnp.float32) * _tile2(scale)
                    + _tile2(shift), 0.0)

    pad_ref[0:p, :] = jnp.zeros((p, lanes), jnp.bfloat16)
    pad_ref[p + hw:, :] = jnp.zeros((p, lanes), jnp.bfloat16)
    pad_ref[p:p + hw, :] = a.astype(jnp.bfloat16)

    w2d = _block_diag2(w2_ref[...].astype(jnp.bfloat16))
    acc = jnp.zeros((hw, lanes), jnp.float32)
    for k in range(9):
        dy, dx = k // 3 - 1, k % 3 - 1
        tap = pad_ref[p + dy * w + dx:p + dy * w + dx + hw, :]
        if dx == -1:
            tap = tap * mask_ref[:, 0:1]
        elif dx == 1:
            tap = tap * mask_ref[:, 1:2]
        acc = acc + jnp.dot(tap, w2d[k],
                            preferred_element_type=jnp.float32)
    t2_ref[0] = acc.astype(jnp.bfloat16)
    s_ref[0] = _stats_rows(acc)


def _conv3_stats_kernel(count, t2_ref, st2_ref, g2_ref, b2_ref, w3_ref,
                        s_ref):
    scale, shift = _fold_bn(st2_ref[...], g2_ref[...], b2_ref[...], count,
                            paired=True)
    a = jnp.maximum(t2_ref[0].astype(jnp.float32) * _tile2(scale)
                    + _tile2(shift), 0.0)
    w3d = _block_diag2(w3_ref[...].astype(jnp.bfloat16)[None])[0]
    y = jnp.dot(a.astype(jnp.bfloat16), w3d,
                preferred_element_type=jnp.float32)
    s_ref[0] = _stats_rows(y)


def _conv3_apply_kernel(cmid, count, t2_ref, st2_ref, g2_ref, b2_ref,
                        w3_ref, st3_ref, g3_ref, b3_ref, x_ref, o_ref):
    scale2, shift2 = _fold_bn(st2_ref[...], g2_ref[...], b2_ref[...], count,
                              paired=True)
    scale3, shift3 = _fold_bn(st3_ref[...], g3_ref[...], b3_ref[...], count,
                              paired=True)
    a = jnp.maximum(t2_ref[0].astype(jnp.float32) * _tile2(scale2)
                    + _tile2(shift2), 0.0)
    ab = a.astype(jnp.bfloat16)
    a2 = jnp.concatenate([ab[:, :cmid], ab[:, cmid:]], axis=0)
    y = jnp.dot(a2, w3_ref[...].astype(jnp.bfloat16),
                preferred_element_type=jnp.float32)
    o_ref[...] = jnp.maximum(y * scale3 + shift3 + x_ref[...], 0.0)


def kernel(x, w1, g1, b1, w2, g2, b2, w3, g3, b3):
    n, cin, h, w = x.shape
    cmid = w1.shape[1]
    cout = w3.shape[1]
    nhw = n * h * w
    npair = n // 2
    hw = h * w
    count = float(nhw)
    cp = _compiler_params()

    x_flat = jnp.transpose(x, (0, 2, 3, 1)).reshape(nhw, cin)

    col = np.arange(hw, dtype=np.int32) % w
    masks = jnp.asarray(
        np.stack([(col != 0), (col != w - 1)], axis=1).astype(np.float32),
        dtype=jnp.bfloat16)

    aff_mid = pl.BlockSpec((1, cmid), lambda i: (0, 0))
    st1_full = pl.BlockSpec((n, 2, cmid), lambda i: (0, 0, 0))
    st2_full = pl.BlockSpec((npair, 2, 2 * cmid), lambda i: (0, 0, 0))

    hw2 = hw // 2
    t1, s1 = pl.pallas_call(
        _conv1_kernel,
        grid=(npair, 2),
        in_specs=[pl.BlockSpec((hw2, cin), lambda i, j: (4 * i + j, 0)),
                  pl.BlockSpec((hw2, cin), lambda i, j: (4 * i + 2 + j, 0)),
                  pl.BlockSpec((cin, cmid), lambda i, j: (0, 0))],
        out_specs=(pl.BlockSpec((1, hw2, 2 * cmid),
                                lambda i, j: (i, j, 0)),
                   pl.BlockSpec((1, 2, cmid), lambda i, j: (2 * i + j, 0, 0))),
        out_shape=(jax.ShapeDtypeStruct((npair, hw, 2 * cmid),
                                        jnp.bfloat16),
                   jax.ShapeDtypeStruct((n, 2, cmid), jnp.float32)),
        compiler_params=_compiler_params(2),
    )(x_flat, x_flat, w1)

    t2, s2 = pl.pallas_call(
        functools.partial(_conv2_kernel, w, count),
        grid=(npair,),
        in_specs=[pl.BlockSpec((1, hw, 2 * cmid), lambda i: (i, 0, 0)),
                  st1_full, aff_mid, aff_mid,
                  pl.BlockSpec((9, cmid, cmid), lambda i: (0, 0, 0)),
                  pl.BlockSpec((hw, 2), lambda i: (0, 0))],
        out_specs=(pl.BlockSpec((1, hw, 2 * cmid), lambda i: (i, 0, 0)),
                   pl.BlockSpec((1, 2, 2 * cmid), lambda i: (i, 0, 0))),
        out_shape=(jax.ShapeDtypeStruct((npair, hw, 2 * cmid),
                                        jnp.bfloat16),
                   jax.ShapeDtypeStruct((npair, 2, 2 * cmid), jnp.float32)),
        scratch_shapes=[pltpu.VMEM((hw + 2 * _HALO, 2 * cmid),
                                   jnp.bfloat16)],
        compiler_params=cp,
    )(t1, s1, g1, b1, w2, masks)

    s3 = pl.pallas_call(
        functools.partial(_conv3_stats_kernel, count),
        grid=(npair,),
        in_specs=[pl.BlockSpec((1, hw, 2 * cmid), lambda i: (i, 0, 0)),
                  st2_full, aff_mid, aff_mid,
                  pl.BlockSpec((cmid, cout), lambda i: (0, 0))],
        out_specs=pl.BlockSpec((1, 2, 2 * cout), lambda i: (i, 0, 0)),
        out_shape=jax.ShapeDtypeStruct((npair, 2, 2 * cout), jnp.float32),
        compiler_params=cp,
    )(t2, s2, g2, b2, w3)

    out = pl.pallas_call(
        functools.partial(_conv3_apply_kernel, cmid, count),
        grid=(npair,),
        in_specs=[pl.BlockSpec((1, hw, 2 * cmid), lambda i: (i, 0, 0)),
                  st2_full, aff_mid, aff_mid,
                  pl.BlockSpec((cmid, cout), lambda i: (0, 0)),
                  pl.BlockSpec((npair, 2, 2 * cout), lambda i: (0, 0, 0)),
                  pl.BlockSpec((1, cout), lambda i: (0, 0)),
                  pl.BlockSpec((1, cout), lambda i: (0, 0)),
                  pl.BlockSpec((2 * hw, cin), lambda i: (i, 0))],
        out_specs=pl.BlockSpec((2 * hw, cout), lambda i: (i, 0)),
        out_shape=jax.ShapeDtypeStruct((nhw, cout), jnp.float32),
        compiler_params=cp,
    )(t2, s2, g2, b2, w3, s3, g3, b3, x_flat)

    return jnp.transpose(out.reshape(n, h, w, cout), (0, 3, 1, 2))


# speedup vs baseline: 1.0515x; 1.0283x over previous
import functools

import jax
import jax.numpy as jnp
import numpy as np
from jax.experimental import pallas as pl
from jax.experimental.pallas import tpu as pltpu

EPS = 1e-5
_HALO = 64


def _compiler_params(ndims=1):
    return pltpu.CompilerParams(
        dimension_semantics=("parallel",) + ("arbitrary",) * (ndims - 1),
        vmem_limit_bytes=64 * 1024 * 1024,
    )


def _row_tile(nhw, target):
    for t in range(min(target, nhw), 7, -1):
        if nhw % t == 0 and t % 8 == 0:
            return t
    return nhw


def _stats_rows(y):
    return jnp.concatenate(
        [jnp.sum(y, axis=0, keepdims=True),
         jnp.sum(y * y, axis=0, keepdims=True)], axis=0)


def _fold_bn(stats, g, b, count, paired=False):
    s = jnp.sum(stats, axis=0)
    if paired:
        c = s.shape[1] // 2
        s = s[:, :c] + s[:, c:]
    mean = s[0:1] / count
    var = jnp.maximum(s[1:2] / count - mean * mean, 0.0)
    scale = g * jax.lax.rsqrt(var + EPS)
    shift = b - mean * scale
    return scale, shift


def _tile2(v):
    return jnp.concatenate([v, v], axis=1)


def _block_diag2(m):
    z = jnp.zeros(m.shape, m.dtype)
    return jnp.concatenate(
        [jnp.concatenate([m, z], axis=2),
         jnp.concatenate([z, m], axis=2)], axis=1)


def _conv1_kernel(hw, x_ref, w1_ref, t1_ref, s_ref):
    y = jnp.dot(x_ref[...].astype(jnp.bfloat16),
                w1_ref[...].astype(jnp.bfloat16),
                preferred_element_type=jnp.float32)
    s_ref[0] = _stats_rows(y)
    yb = y.astype(jnp.bfloat16)
    t1_ref[0] = jnp.concatenate([yb[:hw], yb[hw:]], axis=1)


def _conv2_kernel(w, count, t1_ref, st1_ref, g1_ref, b1_ref, w2_ref,
                  mask_ref, t2_ref, s_ref, pad_ref):
    hw, lanes = t1_ref.shape[1], t1_ref.shape[2]
    p = _HALO
    scale, shift = _fold_bn(st1_ref[...], g1_ref[...], b1_ref[...], count)
    a = jnp.maximum(t1_ref[0].astype(jnp.float32) * _tile2(scale)
                    + _tile2(shift), 0.0)

    pad_ref[0:p, :] = jnp.zeros((p, lanes), jnp.bfloat16)
    pad_ref[p + hw:, :] = jnp.zeros((p, lanes), jnp.bfloat16)
    pad_ref[p:p + hw, :] = a.astype(jnp.bfloat16)

    w2d = _block_diag2(w2_ref[...].astype(jnp.bfloat16))
    acc = jnp.zeros((hw, lanes), jnp.float32)
    for k in range(9):
        dy, dx = k // 3 - 1, k % 3 - 1
        tap = pad_ref[p + dy * w + dx:p + dy * w + dx + hw, :]
        if dx == -1:
            tap = tap * mask_ref[:, 0:1]
        elif dx == 1:
            tap = tap * mask_ref[:, 1:2]
        acc = acc + jnp.dot(tap, w2d[k],
                            preferred_element_type=jnp.float32)
    t2_ref[0] = acc.astype(jnp.bfloat16)
    s_ref[0] = _stats_rows(acc)


def _conv3_stats_kernel(count, t2_ref, st2_ref, g2_ref, b2_ref, w3_ref,
                        s_ref):
    scale, shift = _fold_bn(st2_ref[...], g2_ref[...], b2_ref[...], count,
                            paired=True)
    a = jnp.maximum(t2_ref[0].astype(jnp.float32) * _tile2(scale)
                    + _tile2(shift), 0.0)
    w3d = _block_diag2(w3_ref[...].astype(jnp.bfloat16)[None])[0]
    y = jnp.dot(a.astype(jnp.bfloat16), w3d,
                preferred_element_type=jnp.float32)
    s_ref[0] = _stats_rows(y)


def _conv3_apply_kernel(cmid, count, t2_ref, st2_ref, g2_ref, b2_ref,
                        w3_ref, st3_ref, g3_ref, b3_ref, x_ref, o_ref):
    scale2, shift2 = _fold_bn(st2_ref[...], g2_ref[...], b2_ref[...], count,
                              paired=True)
    scale3, shift3 = _fold_bn(st3_ref[...], g3_ref[...], b3_ref[...], count,
                              paired=True)
    a = jnp.maximum(t2_ref[0].astype(jnp.float32) * _tile2(scale2)
                    + _tile2(shift2), 0.0)
    ab = a.astype(jnp.bfloat16)
    a2 = jnp.concatenate([ab[:, :cmid], ab[:, cmid:]], axis=0)
    y = jnp.dot(a2, w3_ref[...].astype(jnp.bfloat16),
                preferred_element_type=jnp.float32)
    o_ref[...] = jnp.maximum(y * scale3 + shift3 + x_ref[...], 0.0)


def kernel(x, w1, g1, b1, w2, g2, b2, w3, g3, b3):
    n, cin, h, w = x.shape
    cmid = w1.shape[1]
    cout = w3.shape[1]
    nhw = n * h * w
    npair = n // 2
    hw = h * w
    count = float(nhw)
    cp = _compiler_params()

    x_flat = jnp.transpose(x, (0, 2, 3, 1)).reshape(nhw, cin)

    col = np.arange(hw, dtype=np.int32) % w
    masks = jnp.asarray(
        np.stack([(col != 0), (col != w - 1)], axis=1).astype(np.float32),
        dtype=jnp.bfloat16)

    aff_mid = pl.BlockSpec((1, cmid), lambda i: (0, 0))
    st1_full = pl.BlockSpec((npair, 2, cmid), lambda i: (0, 0, 0))
    st2_full = pl.BlockSpec((npair, 2, 2 * cmid), lambda i: (0, 0, 0))

    t1, s1 = pl.pallas_call(
        functools.partial(_conv1_kernel, hw),
        grid=(npair,),
        in_specs=[pl.BlockSpec((2 * hw, cin), lambda i: (i, 0)),
                  pl.BlockSpec((cin, cmid), lambda i: (0, 0))],
        out_specs=(pl.BlockSpec((1, hw, 2 * cmid), lambda i: (i, 0, 0)),
                   pl.BlockSpec((1, 2, cmid), lambda i: (i, 0, 0))),
        out_shape=(jax.ShapeDtypeStruct((npair, hw, 2 * cmid),
                                        jnp.bfloat16),
                   jax.ShapeDtypeStruct((npair, 2, cmid), jnp.float32)),
        compiler_params=cp,
    )(x_flat, w1)

    t2, s2 = pl.pallas_call(
        functools.partial(_conv2_kernel, w, count),
        grid=(npair,),
        in_specs=[pl.BlockSpec((1, hw, 2 * cmid), lambda i: (i, 0, 0)),
                  st1_full, aff_mid, aff_mid,
                  pl.BlockSpec((9, cmid, cmid), lambda i: (0, 0, 0)),
                  pl.BlockSpec((hw, 2), lambda i: (0, 0))],
        out_specs=(pl.BlockSpec((1, hw, 2 * cmid), lambda i: (i, 0, 0)),
                   pl.BlockSpec((1, 2, 2 * cmid), lambda i: (i, 0, 0))),
        out_shape=(jax.ShapeDtypeStruct((npair, hw, 2 * cmid),
                                        jnp.bfloat16),
                   jax.ShapeDtypeStruct((npair, 2, 2 * cmid), jnp.float32)),
        scratch_shapes=[pltpu.VMEM((hw + 2 * _HALO, 2 * cmid),
                                   jnp.bfloat16)],
        compiler_params=cp,
    )(t1, s1, g1, b1, w2, masks)

    s3 = pl.pallas_call(
        functools.partial(_conv3_stats_kernel, count),
        grid=(npair,),
        in_specs=[pl.BlockSpec((1, hw, 2 * cmid), lambda i: (i, 0, 0)),
                  st2_full, aff_mid, aff_mid,
                  pl.BlockSpec((cmid, cout), lambda i: (0, 0))],
        out_specs=pl.BlockSpec((1, 2, 2 * cout), lambda i: (i, 0, 0)),
        out_shape=jax.ShapeDtypeStruct((npair, 2, 2 * cout), jnp.float32),
        compiler_params=cp,
    )(t2, s2, g2, b2, w3)

    out = pl.pallas_call(
        functools.partial(_conv3_apply_kernel, cmid, count),
        grid=(npair,),
        in_specs=[pl.BlockSpec((1, hw, 2 * cmid), lambda i: (i, 0, 0)),
                  st2_full, aff_mid, aff_mid,
                  pl.BlockSpec((cmid, cout), lambda i: (0, 0)),
                  pl.BlockSpec((npair, 2, 2 * cout), lambda i: (0, 0, 0)),
                  pl.BlockSpec((1, cout), lambda i: (0, 0)),
                  pl.BlockSpec((1, cout), lambda i: (0, 0)),
                  pl.BlockSpec((2 * hw, cin), lambda i: (i, 0))],
        out_specs=pl.BlockSpec((2 * hw, cout), lambda i: (i, 0)),
        out_shape=jax.ShapeDtypeStruct((nhw, cout), jnp.float32),
        compiler_params=cp,
    )(t2, s2, g2, b2, w3, s3, g3, b3, x_flat)

    return jnp.transpose(out.reshape(n, h, w, cout), (0, 3, 1, 2))


# speedup vs baseline: 1.1166x; 1.0619x over previous
import functools

import jax
import jax.numpy as jnp
import numpy as np
from jax.experimental import pallas as pl
from jax.experimental.pallas import tpu as pltpu

EPS = 1e-5
_HALO = 64


def _compiler_params(ndims=1):
    return pltpu.CompilerParams(
        dimension_semantics=("parallel",) + ("arbitrary",) * (ndims - 1),
        vmem_limit_bytes=64 * 1024 * 1024,
    )


def _row_tile(nhw, target):
    for t in range(min(target, nhw), 7, -1):
        if nhw % t == 0 and t % 8 == 0:
            return t
    return nhw


def _stats_rows(y):
    return jnp.concatenate(
        [jnp.sum(y, axis=0, keepdims=True),
         jnp.sum(y * y, axis=0, keepdims=True)], axis=0)


def _fold_bn(stats, g, b, count, paired=False):
    s = jnp.sum(stats, axis=0)
    if paired:
        c = s.shape[1] // 2
        s = s[:, :c] + s[:, c:]
    mean = s[0:1] / count
    var = jnp.maximum(s[1:2] / count - mean * mean, 0.0)
    scale = g * jax.lax.rsqrt(var + EPS)
    shift = b - mean * scale
    return scale, shift


def _tile2(v):
    return jnp.concatenate([v, v], axis=1)


def _block_diag2(m):
    z = jnp.zeros(m.shape, m.dtype)
    return jnp.concatenate(
        [jnp.concatenate([m, z], axis=2),
         jnp.concatenate([z, m], axis=2)], axis=1)


def _conv1_kernel(hw, x_ref, w1_ref, t1_ref, s_ref):
    y = jnp.dot(x_ref[...].astype(jnp.bfloat16),
                w1_ref[...].astype(jnp.bfloat16),
                preferred_element_type=jnp.float32)
    s_ref[0] = _stats_rows(y)
    yb = y.astype(jnp.bfloat16)
    t1_ref[0] = jnp.concatenate([yb[:hw], yb[hw:]], axis=1)


def _conv2_kernel(w, count, t1_ref, st1_ref, g1_ref, b1_ref, w2_ref,
                  mask_ref, t2_ref, s_ref, pad_ref):
    hw, lanes = t1_ref.shape[1], t1_ref.shape[2]
    p = _HALO
    scale, shift = _fold_bn(st1_ref[...], g1_ref[...], b1_ref[...], count)
    a = jnp.maximum(t1_ref[0].astype(jnp.float32) * _tile2(scale)
                    + _tile2(shift), 0.0)

    pad_ref[0:p, :] = jnp.zeros((p, lanes), jnp.bfloat16)
    pad_ref[p + hw:, :] = jnp.zeros((p, lanes), jnp.bfloat16)
    pad_ref[p:p + hw, :] = a.astype(jnp.bfloat16)

    w2d = _block_diag2(w2_ref[...].astype(jnp.bfloat16))
    acc = jnp.zeros((hw, lanes), jnp.float32)
    for k in range(9):
        dy, dx = k // 3 - 1, k % 3 - 1
        tap = pad_ref[p + dy * w + dx:p + dy * w + dx + hw, :]
        if dx == -1:
            tap = tap * mask_ref[:, 0:1]
        elif dx == 1:
            tap = tap * mask_ref[:, 1:2]
        acc = acc + jnp.dot(tap, w2d[k],
                            preferred_element_type=jnp.float32)
    t2_ref[0] = acc.astype(jnp.bfloat16)
    s_ref[0] = _stats_rows(acc)


def _conv3_stats_kernel(count, t2_ref, st2_ref, g2_ref, b2_ref, w3_ref,
                        g_ref, cs_ref):
    scale, shift = _fold_bn(st2_ref[...], g2_ref[...], b2_ref[...], count,
                            paired=True)
    a = jnp.maximum(t2_ref[0].astype(jnp.float32) * _tile2(scale)
                    + _tile2(shift), 0.0)
    cs_ref[0] = jnp.sum(a, axis=0, keepdims=True)
    ab = a.astype(jnp.bfloat16)
    g_ref[0] = jax.lax.dot_general(ab, ab, (((0,), (0,)), ((), ())),
                                   preferred_element_type=jnp.float32)


def _conv3_apply_kernel(cmid, count, t2_ref, st2_ref, g2_ref, b2_ref,
                        w3_ref, gram_ref, cs_ref, g3_ref, b3_ref, x_ref,
                        o_ref):
    scale2, shift2 = _fold_bn(st2_ref[...], g2_ref[...], b2_ref[...], count,
                              paired=True)

    g_sum = jnp.sum(gram_ref[...], axis=0)
    g_img = g_sum[:cmid, :cmid] + g_sum[cmid:, cmid:]
    cs = jnp.sum(cs_ref[...], axis=0)
    cs = cs[:, :cmid] + cs[:, cmid:]
    hi = jax.lax.Precision.HIGHEST
    mean3 = jnp.dot(cs, w3_ref[...], precision=hi,
                    preferred_element_type=jnp.float32) / count
    gw = jnp.dot(g_img, w3_ref[...], precision=hi,
                 preferred_element_type=jnp.float32)
    ey2 = jnp.sum(w3_ref[...] * gw, axis=0, keepdims=True) / count
    var3 = jnp.maximum(ey2 - mean3 * mean3, 0.0)
    scale3 = g3_ref[...] * jax.lax.rsqrt(var3 + EPS)
    shift3 = b3_ref[...] - mean3 * scale3

    a = jnp.maximum(t2_ref[0].astype(jnp.float32) * _tile2(scale2)
                    + _tile2(shift2), 0.0)
    ab = a.astype(jnp.bfloat16)
    a2 = jnp.concatenate([ab[:, :cmid], ab[:, cmid:]], axis=0)
    y = jnp.dot(a2, w3_ref[...].astype(jnp.bfloat16),
                preferred_element_type=jnp.float32)
    o_ref[...] = jnp.maximum(y * scale3 + shift3 + x_ref[...], 0.0)


def kernel(x, w1, g1, b1, w2, g2, b2, w3, g3, b3):
    n, cin, h, w = x.shape
    cmid = w1.shape[1]
    cout = w3.shape[1]
    nhw = n * h * w
    npair = n // 2
    hw = h * w
    count = float(nhw)
    cp = _compiler_params()

    x_flat = jnp.transpose(x, (0, 2, 3, 1)).reshape(nhw, cin)

    col = np.arange(hw, dtype=np.int32) % w
    masks = jnp.asarray(
        np.stack([(col != 0), (col != w - 1)], axis=1).astype(np.float32),
        dtype=jnp.bfloat16)

    aff_mid = pl.BlockSpec((1, cmid), lambda i: (0, 0))
    st1_full = pl.BlockSpec((npair, 2, cmid), lambda i: (0, 0, 0))
    st2_full = pl.BlockSpec((npair, 2, 2 * cmid), lambda i: (0, 0, 0))

    t1, s1 = pl.pallas_call(
        functools.partial(_conv1_kernel, hw),
        grid=(npair,),
        in_specs=[pl.BlockSpec((2 * hw, cin), lambda i: (i, 0)),
                  pl.BlockSpec((cin, cmid), lambda i: (0, 0))],
        out_specs=(pl.BlockSpec((1, hw, 2 * cmid), lambda i: (i, 0, 0)),
                   pl.BlockSpec((1, 2, cmid), lambda i: (i, 0, 0))),
        out_shape=(jax.ShapeDtypeStruct((npair, hw, 2 * cmid),
                                        jnp.bfloat16),
                   jax.ShapeDtypeStruct((npair, 2, cmid), jnp.float32)),
        compiler_params=cp,
    )(x_flat, w1)

    t2, s2 = pl.pallas_call(
        functools.partial(_conv2_kernel, w, count),
        grid=(npair,),
        in_specs=[pl.BlockSpec((1, hw, 2 * cmid), lambda i: (i, 0, 0)),
                  st1_full, aff_mid, aff_mid,
                  pl.BlockSpec((9, cmid, cmid), lambda i: (0, 0, 0)),
                  pl.BlockSpec((hw, 2), lambda i: (0, 0))],
        out_specs=(pl.BlockSpec((1, hw, 2 * cmid), lambda i: (i, 0, 0)),
                   pl.BlockSpec((1, 2, 2 * cmid), lambda i: (i, 0, 0))),
        out_shape=(jax.ShapeDtypeStruct((npair, hw, 2 * cmid),
                                        jnp.bfloat16),
                   jax.ShapeDtypeStruct((npair, 2, 2 * cmid), jnp.float32)),
        scratch_shapes=[pltpu.VMEM((hw + 2 * _HALO, 2 * cmid),
                                   jnp.bfloat16)],
        compiler_params=cp,
    )(t1, s1, g1, b1, w2, masks)

    gram, cs = pl.pallas_call(
        functools.partial(_conv3_stats_kernel, count),
        grid=(npair,),
        in_specs=[pl.BlockSpec((1, hw, 2 * cmid), lambda i: (i, 0, 0)),
                  st2_full, aff_mid, aff_mid,
                  pl.BlockSpec((cmid, cout), lambda i: (0, 0))],
        out_specs=(pl.BlockSpec((1, 2 * cmid, 2 * cmid),
                                lambda i: (i, 0, 0)),
                   pl.BlockSpec((1, 1, 2 * cmid), lambda i: (i, 0, 0))),
        out_shape=(jax.ShapeDtypeStruct((npair, 2 * cmid, 2 * cmid),
                                        jnp.float32),
                   jax.ShapeDtypeStruct((npair, 1, 2 * cmid),
                                        jnp.float32)),
        compiler_params=cp,
    )(t2, s2, g2, b2, w3)

    out = pl.pallas_call(
        functools.partial(_conv3_apply_kernel, cmid, count),
        grid=(npair,),
        in_specs=[pl.BlockSpec((1, hw, 2 * cmid), lambda i: (i, 0, 0)),
                  st2_full, aff_mid, aff_mid,
                  pl.BlockSpec((cmid, cout), lambda i: (0, 0)),
                  pl.BlockSpec((npair, 2 * cmid, 2 * cmid),
                               lambda i: (0, 0, 0)),
                  pl.BlockSpec((npair, 1, 2 * cmid), lambda i: (0, 0, 0)),
                  pl.BlockSpec((1, cout), lambda i: (0, 0)),
                  pl.BlockSpec((1, cout), lambda i: (0, 0)),
                  pl.BlockSpec((2 * hw, cin), lambda i: (i, 0))],
        out_specs=pl.BlockSpec((2 * hw, cout), lambda i: (i, 0)),
        out_shape=jax.ShapeDtypeStruct((nhw, cout), jnp.float32),
        compiler_params=cp,
    )(t2, s2, g2, b2, w3, gram, cs, g3, b3, x_flat)

    return jnp.transpose(out.reshape(n, h, w, cout), (0, 3, 1, 2))
